# Initial kernel scaffold; baseline (speedup 1.0000x reference)
import jax
import jax.numpy as jnp
from jax.experimental import pallas as pl


def kernel(x, edge_index, conv1_weight, conv1_bias, conv2_weight, conv2_bias):
    raise NotImplementedError("write your pallas kernel here")


# speedup vs baseline: 16.0891x; 16.0891x over previous
import functools

import jax
import jax.numpy as jnp
from jax import lax
from jax.experimental import pallas as pl
from jax.experimental.pallas import tpu as pltpu
from jax.experimental.pallas import tpu_sc as plsc

CHUNK = 80
NUM_CORES = 2
NUM_SUBCORES = 16
NUM_TILES = NUM_CORES * NUM_SUBCORES


def _mm_body(x_ref, w_ref, o_ref):
  o_ref[...] = lax.dot_general(
      x_ref[...], w_ref[...], (((1,), (0,)), ((), ())),
      precision=lax.Precision.HIGHEST, preferred_element_type=jnp.float32)


def _matmul(x, w):
  n, _ = x.shape
  dout = w.shape[1]
  return pl.pallas_call(
      _mm_body,
      out_shape=jax.ShapeDtypeStruct((n, dout), jnp.float32),
  )(x, w)


def _scale_split_body(deg_ref, mm_ref, g_ref, dis_ref):
  dis = lax.rsqrt(deg_ref[0, :] + deg_ref[1, :])
  g = dis[:, None] * mm_ref[...]
  dh = g.shape[1] // 2
  g_ref[0] = g[:, :dh]
  g_ref[1] = g[:, dh:]
  dis_ref[...] = dis


def _scale_split(deg, mm):
  n, d = mm.shape
  return pl.pallas_call(
      _scale_split_body,
      out_shape=[
          jax.ShapeDtypeStruct((2, n, d // 2), jnp.float32),
          jax.ShapeDtypeStruct((n,), jnp.float32),
      ],
  )(deg, mm)


def _mid_body(acc_ref, dis_ref, b1_ref, w2_ref, g_ref):
  dis = dis_ref[...]
  acc = jnp.concatenate([acc_ref[0], acc_ref[1]], axis=1)
  h = jnp.maximum(dis[:, None] * acc + b1_ref[...][None, :], 0.0)
  g2 = lax.dot_general(
      h, w2_ref[...], (((1,), (0,)), ((), ())),
      precision=lax.Precision.HIGHEST, preferred_element_type=jnp.float32)
  g_ref[...] = dis[:, None] * g2


def _mid_dense(acc1, dis, b1, w2):
  n = dis.shape[0]
  dout = w2.shape[1]
  return pl.pallas_call(
      _mid_body,
      out_shape=jax.ShapeDtypeStruct((n, dout), jnp.float32),
  )(acc1, dis, b1, w2)


def _final_body(acc_ref, dis_ref, b2_ref, o_ref):
  acc = acc_ref[0] + acc_ref[1]
  o_ref[...] = dis_ref[...][:, None] * acc + b2_ref[...][None, :]


def _final(acc2, dis, b2):
  n = dis.shape[0]
  d = b2.shape[0]
  return pl.pallas_call(
      _final_body,
      out_shape=jax.ShapeDtypeStruct((n, d), jnp.float32),
  )(acc2, dis, b2)


def _sc_mesh():
  return plsc.VectorSubcoreMesh(core_axis_name="c", subcore_axis_name="s")


def _hist(col3d, init_deg, ones_chunk):
  n = init_deg.shape[1]
  per_tile = col3d.shape[1]

  @functools.partial(
      pl.kernel,
      out_type=jax.ShapeDtypeStruct((2, n), jnp.float32),
      mesh=_sc_mesh(),
      scratch_types=[
          pltpu.VMEM_SHARED((n,), jnp.float32),
          pltpu.VMEM((per_tile, CHUNK), jnp.int32),
          pltpu.VMEM((CHUNK,), jnp.float32),
      ],
  )
  def hist_kernel(col_hbm, init_hbm, ones_hbm, deg_hbm, deg_sp, col_t, ones_v):
    c = lax.axis_index("c")
    s = lax.axis_index("s")
    pltpu.sync_copy(col_hbm.at[c * NUM_SUBCORES + s], col_t)
    pltpu.sync_copy(ones_hbm, ones_v)

    @pl.when(s == 0)
    def _():
      pltpu.sync_copy(init_hbm.at[c], deg_sp)

    plsc.subcore_barrier()

    @pl.loop(0, per_tile)
    def _(i):
      pltpu.sync_copy(ones_v, deg_sp.at[col_t.at[i]], add=True)

    plsc.subcore_barrier()

    @pl.when(s == 0)
    def _():
      pltpu.sync_copy(deg_sp, deg_hbm.at[c])

  return hist_kernel(col3d, init_deg, ones_chunk)


def _aggregate(g, row3d, col3d):
  _, n, dh = g.shape
  n_blocks, blk_sz = row3d.shape[1], row3d.shape[2]
  rows_lo = (n // NUM_SUBCORES) // 8 * 8
  rows_hi = n - rows_lo * (NUM_SUBCORES - 1)

  @functools.partial(
      pl.kernel,
      out_type=jax.ShapeDtypeStruct((2, n, dh), jnp.float32),
      mesh=_sc_mesh(),
      scratch_types=[
          pltpu.VMEM_SHARED((n, dh), jnp.float32),
          pltpu.VMEM((blk_sz, CHUNK), jnp.int32),
          pltpu.VMEM((blk_sz, CHUNK), jnp.int32),
          pltpu.VMEM((CHUNK, dh), jnp.float32),
          pltpu.SemaphoreType.DMA,
      ],
  )
  def agg_kernel(g_hbm, row_hbm, col_hbm, out_hbm,
                 acc_sp, row_t, col_t, msg_v, gsem):
    c = lax.axis_index("c")
    s = lax.axis_index("s")
    rbase = pl.multiple_of(s * rows_lo, 8)

    @pl.when(s < NUM_SUBCORES - 1)
    def _():
      pltpu.sync_copy(g_hbm.at[c, pl.ds(rbase, rows_lo), :],
                      acc_sp.at[pl.ds(rbase, rows_lo), :])

    @pl.when(s == NUM_SUBCORES - 1)
    def _():
      pltpu.sync_copy(g_hbm.at[c, pl.ds(rbase, rows_hi), :],
                      acc_sp.at[pl.ds(rbase, rows_hi), :])

    plsc.subcore_barrier()

    @pl.loop(0, n_blocks)
    def _(blk):
      pltpu.sync_copy(row_hbm.at[s, blk], row_t)
      pltpu.sync_copy(col_hbm.at[s, blk], col_t)

      @pl.loop(0, blk_sz)
      def _(i):
        pltpu.async_copy(g_hbm.at[c].at[row_t.at[i]], msg_v, gsem).wait()
        pltpu.sync_copy(msg_v, acc_sp.at[col_t.at[i]], add=True)

    plsc.subcore_barrier()

    @pl.when(s < NUM_SUBCORES - 1)
    def _():
      pltpu.sync_copy(acc_sp.at[pl.ds(rbase, rows_lo), :],
                      out_hbm.at[c, pl.ds(rbase, rows_lo), :])

    @pl.when(s == NUM_SUBCORES - 1)
    def _():
      pltpu.sync_copy(acc_sp.at[pl.ds(rbase, rows_hi), :],
                      out_hbm.at[c, pl.ds(rbase, rows_hi), :])

  return agg_kernel(g, row3d, col3d)


def _aggregate_edge_split(g, zeros_init, row5d, col5d):
  n, dh = g.shape
  n_blocks, blk_sz = row5d.shape[2], row5d.shape[3]
  rows_lo = (n // NUM_SUBCORES) // 8 * 8
  rows_hi = n - rows_lo * (NUM_SUBCORES - 1)

  @functools.partial(
      pl.kernel,
      out_type=jax.ShapeDtypeStruct((2, n, dh), jnp.float32),
      mesh=_sc_mesh(),
      scratch_types=[
          pltpu.VMEM_SHARED((n, dh), jnp.float32),
          pltpu.VMEM((blk_sz, CHUNK), jnp.int32),
          pltpu.VMEM((blk_sz, CHUNK), jnp.int32),
          pltpu.VMEM((CHUNK, dh), jnp.float32),
          pltpu.SemaphoreType.DMA,
      ],
  )
  def agg_kernel(g_hbm, z_hbm, row_hbm, col_hbm, out_hbm,
                 acc_sp, row_t, col_t, msg_v, gsem):
    c = lax.axis_index("c")
    s = lax.axis_index("s")
    rbase = pl.multiple_of(s * rows_lo, 8)

    def init_rows(nrows):
      @pl.when(c == 0)
      def _():
        pltpu.sync_copy(g_hbm.at[pl.ds(rbase, nrows), :],
                        acc_sp.at[pl.ds(rbase, nrows), :])

      @pl.when(c == 1)
      def _():
        pltpu.sync_copy(z_hbm.at[pl.ds(rbase, nrows), :],
                        acc_sp.at[pl.ds(rbase, nrows), :])

    @pl.when(s < NUM_SUBCORES - 1)
    def _():
      init_rows(rows_lo)

    @pl.when(s == NUM_SUBCORES - 1)
    def _():
      init_rows(rows_hi)

    plsc.subcore_barrier()

    @pl.loop(0, n_blocks)
    def _(blk):
      pltpu.sync_copy(row_hbm.at[c, s, blk], row_t)
      pltpu.sync_copy(col_hbm.at[c, s, blk], col_t)

      @pl.loop(0, blk_sz)
      def _(i):
        pltpu.async_copy(g_hbm.at[row_t.at[i]], msg_v, gsem).wait()
        pltpu.sync_copy(msg_v, acc_sp.at[col_t.at[i]], add=True)

    plsc.subcore_barrier()

    @pl.when(s < NUM_SUBCORES - 1)
    def _():
      pltpu.sync_copy(acc_sp.at[pl.ds(rbase, rows_lo), :],
                      out_hbm.at[c, pl.ds(rbase, rows_lo), :])

    @pl.when(s == NUM_SUBCORES - 1)
    def _():
      pltpu.sync_copy(acc_sp.at[pl.ds(rbase, rows_hi), :],
                      out_hbm.at[c, pl.ds(rbase, rows_hi), :])

  return agg_kernel(g, zeros_init, row5d, col5d)


def kernel(x, edge_index, conv1_weight, conv1_bias, conv2_weight, conv2_bias):
  n = x.shape[0]
  e = edge_index.shape[1]
  n_chunks = e // CHUNK
  blk_sz = 25
  n_blocks = n_chunks // NUM_SUBCORES // blk_sz
  row3d = edge_index[0].reshape(NUM_SUBCORES, n_blocks, blk_sz, CHUNK)
  col3d = edge_index[1].reshape(NUM_SUBCORES, n_blocks, blk_sz, CHUNK)
  row5d = edge_index[0].reshape(NUM_CORES, NUM_SUBCORES, n_blocks // 2,
                                blk_sz, CHUNK)
  col5d = edge_index[1].reshape(NUM_CORES, NUM_SUBCORES, n_blocks // 2,
                                blk_sz, CHUNK)
  col3d_hist = edge_index[1].reshape(NUM_TILES, n_chunks // NUM_TILES, CHUNK)
  init_deg = jnp.stack([jnp.ones((n,), jnp.float32),
                        jnp.zeros((n,), jnp.float32)])
  ones_chunk = jnp.ones((CHUNK,), jnp.float32)
  zeros_feat = jnp.zeros((n, conv2_weight.shape[1]), jnp.float32)

  deg = _hist(col3d_hist, init_deg, ones_chunk)
  mm1 = _matmul(x, conv1_weight)
  g1, dis = _scale_split(deg, mm1)
  acc1 = _aggregate(g1, row3d, col3d)
  g2 = _mid_dense(acc1, dis, conv1_bias, conv2_weight)
  acc2 = _aggregate_edge_split(g2, zeros_feat, row5d, col5d)
  return _final(acc2, dis, conv2_bias)


# speedup vs baseline: 24.8898x; 1.5470x over previous
import functools

import jax
import jax.numpy as jnp
from jax import lax
from jax.experimental import pallas as pl
from jax.experimental.pallas import tpu as pltpu
from jax.experimental.pallas import tpu_sc as plsc

CHUNK = 80
NUM_CORES = 2
NUM_SUBCORES = 16
NUM_TILES = NUM_CORES * NUM_SUBCORES


def _mm_body(x_ref, w_ref, o_ref):
  o_ref[...] = lax.dot_general(
      x_ref[...], w_ref[...], (((1,), (0,)), ((), ())),
      precision=lax.Precision.HIGHEST, preferred_element_type=jnp.float32)


def _matmul(x, w):
  n, _ = x.shape
  dout = w.shape[1]
  return pl.pallas_call(
      _mm_body,
      out_shape=jax.ShapeDtypeStruct((n, dout), jnp.float32),
  )(x, w)


def _scale_split_body(deg_ref, mm_ref, g_ref, dis_ref):
  dis = lax.rsqrt(deg_ref[0, :] + deg_ref[1, :])
  g = dis[:, None] * mm_ref[...]
  dh = g.shape[1] // 2
  g_ref[0] = g[:, :dh]
  g_ref[1] = g[:, dh:]
  dis_ref[...] = dis


def _scale_split(deg, mm):
  n, d = mm.shape
  return pl.pallas_call(
      _scale_split_body,
      out_shape=[
          jax.ShapeDtypeStruct((2, n, d // 2), jnp.float32),
          jax.ShapeDtypeStruct((n,), jnp.float32),
      ],
  )(deg, mm)


def _mid_body(acc_ref, dis_ref, b1_ref, w2_ref, g_ref):
  dis = dis_ref[...]
  acc = jnp.concatenate([acc_ref[0], acc_ref[1]], axis=1)
  h = jnp.maximum(dis[:, None] * acc + b1_ref[...][None, :], 0.0)
  g2 = lax.dot_general(
      h, w2_ref[...], (((1,), (0,)), ((), ())),
      precision=lax.Precision.HIGHEST, preferred_element_type=jnp.float32)
  g_ref[...] = dis[:, None] * g2


def _mid_dense(acc1, dis, b1, w2):
  n = dis.shape[0]
  dout = w2.shape[1]
  return pl.pallas_call(
      _mid_body,
      out_shape=jax.ShapeDtypeStruct((n, dout), jnp.float32),
  )(acc1, dis, b1, w2)


def _final_body(acc_ref, dis_ref, b2_ref, o_ref):
  acc = acc_ref[0] + acc_ref[1]
  o_ref[...] = dis_ref[...][:, None] * acc + b2_ref[...][None, :]


def _final(acc2, dis, b2):
  n = dis.shape[0]
  d = b2.shape[0]
  return pl.pallas_call(
      _final_body,
      out_shape=jax.ShapeDtypeStruct((n, d), jnp.float32),
  )(acc2, dis, b2)


def _sc_mesh():
  return plsc.VectorSubcoreMesh(core_axis_name="c", subcore_axis_name="s")


def _hist(col3d, init_deg, ones_chunk):
  n = init_deg.shape[1]
  per_tile = col3d.shape[1]

  @functools.partial(
      pl.kernel,
      out_type=jax.ShapeDtypeStruct((2, n), jnp.float32),
      mesh=_sc_mesh(),
      scratch_types=[
          pltpu.VMEM_SHARED((n,), jnp.float32),
          pltpu.VMEM((per_tile, CHUNK), jnp.int32),
          pltpu.VMEM((CHUNK,), jnp.float32),
      ],
  )
  def hist_kernel(col_hbm, init_hbm, ones_hbm, deg_hbm, deg_sp, col_t, ones_v):
    c = lax.axis_index("c")
    s = lax.axis_index("s")
    pltpu.sync_copy(col_hbm.at[c * NUM_SUBCORES + s], col_t)
    pltpu.sync_copy(ones_hbm, ones_v)

    @pl.when(s == 0)
    def _():
      pltpu.sync_copy(init_hbm.at[c], deg_sp)

    plsc.subcore_barrier()

    @pl.loop(0, per_tile)
    def _(i):
      pltpu.sync_copy(ones_v, deg_sp.at[col_t.at[i]], add=True)

    plsc.subcore_barrier()

    @pl.when(s == 0)
    def _():
      pltpu.sync_copy(deg_sp, deg_hbm.at[c])

  return hist_kernel(col3d, init_deg, ones_chunk)


def _aggregate(g, row3d, col3d):
  _, n, dh = g.shape
  n_blocks, blk_sz = row3d.shape[1], row3d.shape[2]
  rows_lo = (n // NUM_SUBCORES) // 8 * 8
  rows_hi = n - rows_lo * (NUM_SUBCORES - 1)

  @functools.partial(
      pl.kernel,
      out_type=jax.ShapeDtypeStruct((2, n, dh), jnp.float32),
      mesh=_sc_mesh(),
      scratch_types=[
          pltpu.VMEM_SHARED((n, dh), jnp.float32),
          pltpu.VMEM((blk_sz, CHUNK), jnp.int32),
          pltpu.VMEM((blk_sz, CHUNK), jnp.int32),
          pltpu.VMEM((2, CHUNK, dh), jnp.float32),
          pltpu.SemaphoreType.DMA((2,)),
      ],
  )
  def agg_kernel(g_hbm, row_hbm, col_hbm, out_hbm,
                 acc_sp, row_t, col_t, msg_v, gsem):
    c = lax.axis_index("c")
    s = lax.axis_index("s")
    rbase = pl.multiple_of(s * rows_lo, 8)

    @pl.when(s < NUM_SUBCORES - 1)
    def _():
      pltpu.sync_copy(g_hbm.at[c, pl.ds(rbase, rows_lo), :],
                      acc_sp.at[pl.ds(rbase, rows_lo), :])

    @pl.when(s == NUM_SUBCORES - 1)
    def _():
      pltpu.sync_copy(g_hbm.at[c, pl.ds(rbase, rows_hi), :],
                      acc_sp.at[pl.ds(rbase, rows_hi), :])

    plsc.subcore_barrier()

    @pl.loop(0, n_blocks)
    def _(blk):
      pltpu.sync_copy(row_hbm.at[s, blk], row_t)
      pltpu.sync_copy(col_hbm.at[s, blk], col_t)
      pltpu.async_copy(g_hbm.at[c].at[row_t.at[0]], msg_v.at[0], gsem.at[0])

      @pl.loop(0, blk_sz)
      def _(i):
        b = lax.rem(i, 2)

        @pl.when(i + 1 < blk_sz)
        def _():
          pltpu.async_copy(g_hbm.at[c].at[row_t.at[i + 1]],
                           msg_v.at[lax.rem(i + 1, 2)],
                           gsem.at[lax.rem(i + 1, 2)])

        pltpu.make_async_copy(g_hbm.at[c].at[row_t.at[i]], msg_v.at[b],
                              gsem.at[b]).wait()
        pltpu.sync_copy(msg_v.at[b], acc_sp.at[col_t.at[i]], add=True)

    plsc.subcore_barrier()

    @pl.when(s < NUM_SUBCORES - 1)
    def _():
      pltpu.sync_copy(acc_sp.at[pl.ds(rbase, rows_lo), :],
                      out_hbm.at[c, pl.ds(rbase, rows_lo), :])

    @pl.when(s == NUM_SUBCORES - 1)
    def _():
      pltpu.sync_copy(acc_sp.at[pl.ds(rbase, rows_hi), :],
                      out_hbm.at[c, pl.ds(rbase, rows_hi), :])

  return agg_kernel(g, row3d, col3d)


def _aggregate_edge_split(g, zeros_init, row5d, col5d):
  n, dh = g.shape
  n_blocks, blk_sz = row5d.shape[2], row5d.shape[3]
  rows_lo = (n // NUM_SUBCORES) // 8 * 8
  rows_hi = n - rows_lo * (NUM_SUBCORES - 1)

  @functools.partial(
      pl.kernel,
      out_type=jax.ShapeDtypeStruct((2, n, dh), jnp.float32),
      mesh=_sc_mesh(),
      scratch_types=[
          pltpu.VMEM_SHARED((n, dh), jnp.float32),
          pltpu.VMEM((blk_sz, CHUNK), jnp.int32),
          pltpu.VMEM((blk_sz, CHUNK), jnp.int32),
          pltpu.VMEM((2, CHUNK, dh), jnp.float32),
          pltpu.SemaphoreType.DMA((2,)),
      ],
  )
  def agg_kernel(g_hbm, z_hbm, row_hbm, col_hbm, out_hbm,
                 acc_sp, row_t, col_t, msg_v, gsem):
    c = lax.axis_index("c")
    s = lax.axis_index("s")
    rbase = pl.multiple_of(s * rows_lo, 8)

    def init_rows(nrows):
      @pl.when(c == 0)
      def _():
        pltpu.sync_copy(g_hbm.at[pl.ds(rbase, nrows), :],
                        acc_sp.at[pl.ds(rbase, nrows), :])

      @pl.when(c == 1)
      def _():
        pltpu.sync_copy(z_hbm.at[pl.ds(rbase, nrows), :],
                        acc_sp.at[pl.ds(rbase, nrows), :])

    @pl.when(s < NUM_SUBCORES - 1)
    def _():
      init_rows(rows_lo)

    @pl.when(s == NUM_SUBCORES - 1)
    def _():
      init_rows(rows_hi)

    plsc.subcore_barrier()

    @pl.loop(0, n_blocks)
    def _(blk):
      pltpu.sync_copy(row_hbm.at[c, s, blk], row_t)
      pltpu.sync_copy(col_hbm.at[c, s, blk], col_t)
      pltpu.async_copy(g_hbm.at[row_t.at[0]], msg_v.at[0], gsem.at[0])

      @pl.loop(0, blk_sz)
      def _(i):
        b = lax.rem(i, 2)

        @pl.when(i + 1 < blk_sz)
        def _():
          pltpu.async_copy(g_hbm.at[row_t.at[i + 1]],
                           msg_v.at[lax.rem(i + 1, 2)],
                           gsem.at[lax.rem(i + 1, 2)])

        pltpu.make_async_copy(g_hbm.at[row_t.at[i]], msg_v.at[b],
                              gsem.at[b]).wait()
        pltpu.sync_copy(msg_v.at[b], acc_sp.at[col_t.at[i]], add=True)

    plsc.subcore_barrier()

    @pl.when(s < NUM_SUBCORES - 1)
    def _():
      pltpu.sync_copy(acc_sp.at[pl.ds(rbase, rows_lo), :],
                      out_hbm.at[c, pl.ds(rbase, rows_lo), :])

    @pl.when(s == NUM_SUBCORES - 1)
    def _():
      pltpu.sync_copy(acc_sp.at[pl.ds(rbase, rows_hi), :],
                      out_hbm.at[c, pl.ds(rbase, rows_hi), :])

  return agg_kernel(g, zeros_init, row5d, col5d)


def kernel(x, edge_index, conv1_weight, conv1_bias, conv2_weight, conv2_bias):
  n = x.shape[0]
  e = edge_index.shape[1]
  n_chunks = e // CHUNK
  blk_sz = 25
  n_blocks = n_chunks // NUM_SUBCORES // blk_sz
  row3d = edge_index[0].reshape(NUM_SUBCORES, n_blocks, blk_sz, CHUNK)
  col3d = edge_index[1].reshape(NUM_SUBCORES, n_blocks, blk_sz, CHUNK)
  row5d = edge_index[0].reshape(NUM_CORES, NUM_SUBCORES, n_blocks // 2,
                                blk_sz, CHUNK)
  col5d = edge_index[1].reshape(NUM_CORES, NUM_SUBCORES, n_blocks // 2,
                                blk_sz, CHUNK)
  col3d_hist = edge_index[1].reshape(NUM_TILES, n_chunks // NUM_TILES, CHUNK)
  init_deg = jnp.stack([jnp.ones((n,), jnp.float32),
                        jnp.zeros((n,), jnp.float32)])
  ones_chunk = jnp.ones((CHUNK,), jnp.float32)
  zeros_feat = jnp.zeros((n, conv2_weight.shape[1]), jnp.float32)

  deg = _hist(col3d_hist, init_deg, ones_chunk)
  mm1 = _matmul(x, conv1_weight)
  g1, dis = _scale_split(deg, mm1)
  acc1 = _aggregate(g1, row3d, col3d)
  g2 = _mid_dense(acc1, dis, conv1_bias, conv2_weight)
  acc2 = _aggregate_edge_split(g2, zeros_feat, row5d, col5d)
  return _final(acc2, dis, conv2_bias)


# speedup vs baseline: 24.8974x; 1.0003x over previous
import functools

import jax
import jax.numpy as jnp
from jax import lax
from jax.experimental import pallas as pl
from jax.experimental.pallas import tpu as pltpu
from jax.experimental.pallas import tpu_sc as plsc

CHUNK = 80
NUM_CORES = 2
NUM_SUBCORES = 16
NUM_TILES = NUM_CORES * NUM_SUBCORES


def _mm_body(x_ref, w_ref, o_ref):
  o_ref[...] = lax.dot_general(
      x_ref[...], w_ref[...], (((1,), (0,)), ((), ())),
      precision=lax.Precision.HIGHEST, preferred_element_type=jnp.float32)


def _matmul(x, w):
  n, _ = x.shape
  dout = w.shape[1]
  return pl.pallas_call(
      _mm_body,
      out_shape=jax.ShapeDtypeStruct((n, dout), jnp.float32),
  )(x, w)


def _scale_split_body(deg_ref, mm_ref, g_ref, dis_ref):
  dis = lax.rsqrt(deg_ref[0, :] + deg_ref[1, :])
  g = dis[:, None] * mm_ref[...]
  dh = g.shape[1] // 2
  g_ref[0] = g[:, :dh]
  g_ref[1] = g[:, dh:]
  dis_ref[...] = dis


def _scale_split(deg, mm):
  n, d = mm.shape
  return pl.pallas_call(
      _scale_split_body,
      out_shape=[
          jax.ShapeDtypeStruct((2, n, d // 2), jnp.float32),
          jax.ShapeDtypeStruct((n,), jnp.float32),
      ],
  )(deg, mm)


def _mid_body(acc_ref, dis_ref, b1_ref, w2_ref, g_ref):
  dis = dis_ref[...]
  acc = jnp.concatenate([acc_ref[0], acc_ref[1]], axis=1)
  h = jnp.maximum(dis[:, None] * acc + b1_ref[...][None, :], 0.0)
  g2 = lax.dot_general(
      h, w2_ref[...], (((1,), (0,)), ((), ())),
      precision=lax.Precision.HIGHEST, preferred_element_type=jnp.float32)
  g_ref[...] = dis[:, None] * g2


def _mid_dense(acc1, dis, b1, w2):
  n = dis.shape[0]
  dout = w2.shape[1]
  return pl.pallas_call(
      _mid_body,
      out_shape=jax.ShapeDtypeStruct((n, dout), jnp.float32),
  )(acc1, dis, b1, w2)


def _final_body(acc_ref, dis_ref, b2_ref, o_ref):
  acc = acc_ref[0] + acc_ref[1]
  o_ref[...] = dis_ref[...][:, None] * acc + b2_ref[...][None, :]


def _final(acc2, dis, b2):
  n = dis.shape[0]
  d = b2.shape[0]
  return pl.pallas_call(
      _final_body,
      out_shape=jax.ShapeDtypeStruct((n, d), jnp.float32),
  )(acc2, dis, b2)


def _sc_mesh():
  return plsc.VectorSubcoreMesh(core_axis_name="c", subcore_axis_name="s")


def _edge_stream(gsrc, row_blk, col_blk, n_blocks, blk_sz,
                 acc_sp, row_t, col_t, msg_v, gsem, ssem):
  @pl.loop(0, n_blocks)
  def _(blk):
    pltpu.sync_copy(row_blk(blk), row_t)
    pltpu.sync_copy(col_blk(blk), col_t)
    pltpu.async_copy(gsrc.at[row_t.at[0]], msg_v.at[0], gsem.at[0])

    @pl.loop(0, blk_sz)
    def _(i):
      b = lax.rem(i, 2)
      nb = lax.rem(i + 1, 2)

      @pl.when(i > 0)
      def _():
        pltpu.make_async_copy(msg_v.at[nb], acc_sp.at[col_t.at[i]],
                              ssem.at[nb]).wait()

      @pl.when(i + 1 < blk_sz)
      def _():
        pltpu.async_copy(gsrc.at[row_t.at[i + 1]], msg_v.at[nb], gsem.at[nb])

      pltpu.make_async_copy(gsrc.at[row_t.at[i]], msg_v.at[b],
                            gsem.at[b]).wait()
      pltpu.async_copy(msg_v.at[b], acc_sp.at[col_t.at[i]], ssem.at[b],
                       add=True)

    last = (blk_sz - 1) % 2
    pltpu.make_async_copy(msg_v.at[last], acc_sp.at[col_t.at[blk_sz - 1]],
                          ssem.at[last]).wait()


def _hist(col3d, init_deg, ones_chunk):
  n = init_deg.shape[1]
  per_tile = col3d.shape[1]

  @functools.partial(
      pl.kernel,
      out_type=jax.ShapeDtypeStruct((2, n), jnp.float32),
      mesh=_sc_mesh(),
      scratch_types=[
          pltpu.VMEM_SHARED((n,), jnp.float32),
          pltpu.VMEM((per_tile, CHUNK), jnp.int32),
          pltpu.VMEM((CHUNK,), jnp.float32),
      ],
  )
  def hist_kernel(col_hbm, init_hbm, ones_hbm, deg_hbm, deg_sp, col_t, ones_v):
    c = lax.axis_index("c")
    s = lax.axis_index("s")
    pltpu.sync_copy(col_hbm.at[c * NUM_SUBCORES + s], col_t)
    pltpu.sync_copy(ones_hbm, ones_v)

    @pl.when(s == 0)
    def _():
      pltpu.sync_copy(init_hbm.at[c], deg_sp)

    plsc.subcore_barrier()

    @pl.loop(0, per_tile)
    def _(i):
      pltpu.sync_copy(ones_v, deg_sp.at[col_t.at[i]], add=True)

    plsc.subcore_barrier()

    @pl.when(s == 0)
    def _():
      pltpu.sync_copy(deg_sp, deg_hbm.at[c])

  return hist_kernel(col3d, init_deg, ones_chunk)


def _aggregate(g, row3d, col3d):
  _, n, dh = g.shape
  n_blocks, blk_sz = row3d.shape[1], row3d.shape[2]
  rows_lo = (n // NUM_SUBCORES) // 8 * 8
  rows_hi = n - rows_lo * (NUM_SUBCORES - 1)

  @functools.partial(
      pl.kernel,
      out_type=jax.ShapeDtypeStruct((2, n, dh), jnp.float32),
      mesh=_sc_mesh(),
      scratch_types=[
          pltpu.VMEM_SHARED((n, dh), jnp.float32),
          pltpu.VMEM((blk_sz, CHUNK), jnp.int32),
          pltpu.VMEM((blk_sz, CHUNK), jnp.int32),
          pltpu.VMEM((2, CHUNK, dh), jnp.float32),
          pltpu.SemaphoreType.DMA((2,)),
          pltpu.SemaphoreType.DMA((2,)),
      ],
  )
  def agg_kernel(g_hbm, row_hbm, col_hbm, out_hbm,
                 acc_sp, row_t, col_t, msg_v, gsem, ssem):
    c = lax.axis_index("c")
    s = lax.axis_index("s")
    rbase = pl.multiple_of(s * rows_lo, 8)

    @pl.when(s < NUM_SUBCORES - 1)
    def _():
      pltpu.sync_copy(g_hbm.at[c, pl.ds(rbase, rows_lo), :],
                      acc_sp.at[pl.ds(rbase, rows_lo), :])

    @pl.when(s == NUM_SUBCORES - 1)
    def _():
      pltpu.sync_copy(g_hbm.at[c, pl.ds(rbase, rows_hi), :],
                      acc_sp.at[pl.ds(rbase, rows_hi), :])

    plsc.subcore_barrier()

    _edge_stream(g_hbm.at[c], lambda blk: row_hbm.at[s, blk],
                 lambda blk: col_hbm.at[s, blk], n_blocks, blk_sz,
                 acc_sp, row_t, col_t, msg_v, gsem, ssem)

    plsc.subcore_barrier()

    @pl.when(s < NUM_SUBCORES - 1)
    def _():
      pltpu.sync_copy(acc_sp.at[pl.ds(rbase, rows_lo), :],
                      out_hbm.at[c, pl.ds(rbase, rows_lo), :])

    @pl.when(s == NUM_SUBCORES - 1)
    def _():
      pltpu.sync_copy(acc_sp.at[pl.ds(rbase, rows_hi), :],
                      out_hbm.at[c, pl.ds(rbase, rows_hi), :])

  return agg_kernel(g, row3d, col3d)


def _aggregate_edge_split(g, zeros_init, row5d, col5d):
  n, dh = g.shape
  n_blocks, blk_sz = row5d.shape[2], row5d.shape[3]
  rows_lo = (n // NUM_SUBCORES) // 8 * 8
  rows_hi = n - rows_lo * (NUM_SUBCORES - 1)

  @functools.partial(
      pl.kernel,
      out_type=jax.ShapeDtypeStruct((2, n, dh), jnp.float32),
      mesh=_sc_mesh(),
      scratch_types=[
          pltpu.VMEM_SHARED((n, dh), jnp.float32),
          pltpu.VMEM((blk_sz, CHUNK), jnp.int32),
          pltpu.VMEM((blk_sz, CHUNK), jnp.int32),
          pltpu.VMEM((2, CHUNK, dh), jnp.float32),
          pltpu.SemaphoreType.DMA((2,)),
          pltpu.SemaphoreType.DMA((2,)),
      ],
  )
  def agg_kernel(g_hbm, z_hbm, row_hbm, col_hbm, out_hbm,
                 acc_sp, row_t, col_t, msg_v, gsem, ssem):
    c = lax.axis_index("c")
    s = lax.axis_index("s")
    rbase = pl.multiple_of(s * rows_lo, 8)

    def init_rows(nrows):
      @pl.when(c == 0)
      def _():
        pltpu.sync_copy(g_hbm.at[pl.ds(rbase, nrows), :],
                        acc_sp.at[pl.ds(rbase, nrows), :])

      @pl.when(c == 1)
      def _():
        pltpu.sync_copy(z_hbm.at[pl.ds(rbase, nrows), :],
                        acc_sp.at[pl.ds(rbase, nrows), :])

    @pl.when(s < NUM_SUBCORES - 1)
    def _():
      init_rows(rows_lo)

    @pl.when(s == NUM_SUBCORES - 1)
    def _():
      init_rows(rows_hi)

    plsc.subcore_barrier()

    _edge_stream(g_hbm, lambda blk: row_hbm.at[c, s, blk],
                 lambda blk: col_hbm.at[c, s, blk], n_blocks, blk_sz,
                 acc_sp, row_t, col_t, msg_v, gsem, ssem)

    plsc.subcore_barrier()

    @pl.when(s < NUM_SUBCORES - 1)
    def _():
      pltpu.sync_copy(acc_sp.at[pl.ds(rbase, rows_lo), :],
                      out_hbm.at[c, pl.ds(rbase, rows_lo), :])

    @pl.when(s == NUM_SUBCORES - 1)
    def _():
      pltpu.sync_copy(acc_sp.at[pl.ds(rbase, rows_hi), :],
                      out_hbm.at[c, pl.ds(rbase, rows_hi), :])

  return agg_kernel(g, zeros_init, row5d, col5d)


def kernel(x, edge_index, conv1_weight, conv1_bias, conv2_weight, conv2_bias):
  n = x.shape[0]
  e = edge_index.shape[1]
  n_chunks = e // CHUNK
  blk_sz = 25
  n_blocks = n_chunks // NUM_SUBCORES // blk_sz
  row3d = edge_index[0].reshape(NUM_SUBCORES, n_blocks, blk_sz, CHUNK)
  col3d = edge_index[1].reshape(NUM_SUBCORES, n_blocks, blk_sz, CHUNK)
  row5d = edge_index[0].reshape(NUM_CORES, NUM_SUBCORES, n_blocks // 2,
                                blk_sz, CHUNK)
  col5d = edge_index[1].reshape(NUM_CORES, NUM_SUBCORES, n_blocks // 2,
                                blk_sz, CHUNK)
  col3d_hist = edge_index[1].reshape(NUM_TILES, n_chunks // NUM_TILES, CHUNK)
  init_deg = jnp.stack([jnp.ones((n,), jnp.float32),
                        jnp.zeros((n,), jnp.float32)])
  ones_chunk = jnp.ones((CHUNK,), jnp.float32)
  zeros_feat = jnp.zeros((n, conv2_weight.shape[1]), jnp.float32)

  deg = _hist(col3d_hist, init_deg, ones_chunk)
  mm1 = _matmul(x, conv1_weight)
  g1, dis = _scale_split(deg, mm1)
  acc1 = _aggregate(g1, row3d, col3d)
  g2 = _mid_dense(acc1, dis, conv1_bias, conv2_weight)
  acc2 = _aggregate_edge_split(g2, zeros_feat, row5d, col5d)
  return _final(acc2, dis, conv2_bias)
